# baseline (speedup 1.0000x reference)
import functools

import jax
import jax.numpy as jnp
from jax import lax
from jax.experimental import pallas as pl
from jax.experimental.pallas import tpu as pltpu
from jax.experimental.pallas import tpu_sc as plsc

D = 128
VOCAB = 5000
NC, NS = 2, 16
NW = NC * NS
BATCH, HIST = 4096, 200
B = BATCH * HIST
B_PER_W = B // NW
GROW = 64
CHUNK = 2 * GROW
NCH = B_PER_W // CHUNK
NIR = NCH

_mesh = plsc.VectorSubcoreMesh(core_axis_name="c", subcore_axis_name="s")


@functools.partial(
    pl.kernel,
    mesh=_mesh,
    out_type=jax.ShapeDtypeStruct((B, D), jnp.float32),
    scratch_types=[
        pltpu.VMEM((NIR, CHUNK), jnp.int32),
        pltpu.VMEM((CHUNK, D), jnp.float32),
        pltpu.VMEM((CHUNK, D), jnp.float32),
        pltpu.VMEM((CHUNK, D), jnp.float32),
        pltpu.VMEM_SHARED((VOCAB, D), jnp.float32),
        pltpu.SemaphoreType.DMA,
        pltpu.SemaphoreType.DMA,
        pltpu.SemaphoreType.DMA,
        pltpu.SemaphoreType.DMA,
        pltpu.SemaphoreType.DMA,
        pltpu.SemaphoreType.DMA,
        pltpu.SemaphoreType.DMA,
        pltpu.SemaphoreType.DMA,
        pltpu.SemaphoreType.DMA,
        pltpu.SemaphoreType.DMA,
        pltpu.SemaphoreType.DMA,
        pltpu.SemaphoreType.DMA,
    ],
)
def _emb_lookup(x_hbm, table_hbm, out_hbm, idx_v,
                rows0, rows1, rows2, table_sh,
                g00, g01, g10, g11, g20, g21,
                w00, w01, w10, w11, w20, w21):
    rows = (rows0, rows1, rows2)
    gsem = ((g00, g01), (g10, g11), (g20, g21))
    wsem = ((w00, w01), (w10, w11), (w20, w21))

    wid = lax.axis_index("s") * NC + lax.axis_index("c")
    base = wid * B_PER_W

    sid = lax.axis_index("s")
    pltpu.sync_copy(table_hbm.at[pl.ds(312 * sid, 320)],
                    table_sh.at[pl.ds(312 * sid, 320)])
    pltpu.sync_copy(x_hbm.at[pl.ds(wid * NIR, NIR)], idx_v)
    plsc.subcore_barrier()

    def fire_gather(c, b, h):
        pltpu.async_copy(table_sh.at[idx_v.at[c, pl.ds(h * GROW, GROW)]],
                         rows[b].at[pl.ds(h * GROW, GROW)], gsem[b][h])

    def drain_gather(b, h):
        pltpu.make_async_copy(table_sh.at[idx_v.at[0, pl.ds(0, GROW)]],
                              rows[b].at[pl.ds(h * GROW, GROW)],
                              gsem[b][h]).wait()

    def fire_write(i, b, h):
        pltpu.async_copy(rows[b].at[pl.ds(h * GROW, GROW)],
                         out_hbm.at[pl.ds(base + i * CHUNK + h * GROW, GROW)],
                         wsem[b][h])

    def drain_write(b, h):
        pltpu.make_async_copy(rows[b].at[pl.ds(h * GROW, GROW)],
                              out_hbm.at[pl.ds(base, GROW)],
                              wsem[b][h]).wait()

    def fire_gather2(c, b):
        fire_gather(c, b, 0)
        fire_gather(c, b, 1)

    fire_gather2(0, 0)
    fire_gather2(1, 1)

    fire_gather2(2, 2)
    drain_gather(0, 0)
    fire_write(0, 0, 0)
    drain_gather(0, 1)
    fire_write(0, 0, 1)
    drain_write(0, 0)
    drain_write(0, 1)
    fire_gather2(3, 0)
    drain_gather(1, 0)
    fire_write(1, 1, 0)
    drain_gather(1, 1)
    fire_write(1, 1, 1)
    drain_write(1, 0)
    drain_write(1, 1)
    fire_gather2(4, 1)
    drain_gather(2, 0)
    fire_write(2, 2, 0)
    drain_gather(2, 1)
    fire_write(2, 2, 1)

    def sub_iter(i, b):
        tb = (b + 2) % 3
        drain_gather(b, 0)
        fire_write(i, b, 0)
        drain_gather(b, 1)
        fire_write(i, b, 1)
        drain_write(tb, 0)
        drain_write(tb, 1)
        fire_gather2(i + 2, tb)

    def body(g, carry):
        for b in range(3):
            sub_iter(3 * g + b, b)
        return carry

    lax.fori_loop(1, 66, body, None)

    for i in (198, 199):
        b = i % 3
        tb = (b + 2) % 3
        drain_gather(b, 0)
        fire_write(i, b, 0)
        drain_gather(b, 1)
        fire_write(i, b, 1)
        drain_write(tb, 0)
        drain_write(tb, 1)
    drain_write(199 % 3, 0)
    drain_write(199 % 3, 1)


def kernel(x, table):
    x2 = x.reshape(NW * NIR, CHUNK).astype(jnp.int32)
    out = _emb_lookup(x2, table)
    return out.reshape(BATCH, HIST, D)
